# Initial kernel scaffold; baseline (speedup 1.0000x reference)
import jax
import jax.numpy as jnp
from jax.experimental import pallas as pl


def kernel(x):
    raise NotImplementedError("write your pallas kernel here")


# speedup vs baseline: 10608.5393x; 10608.5393x over previous
import jax
import jax.numpy as jnp
from jax.experimental import pallas as pl

_COLS = 2048
_T0 = 1.0 / 256.0
_T1 = 2.0 / 256.0


def _count_body(x_ref, c0_ref, c1_ref):
    i = pl.program_id(0)
    v = x_ref[...]
    p0 = jnp.sum((v < _T0).astype(jnp.float32), axis=0, keepdims=True)
    p1 = jnp.sum((v < _T1).astype(jnp.float32), axis=0, keepdims=True)

    @pl.when(i == 0)
    def _():
        c0_ref[...] = p0
        c1_ref[...] = p1

    @pl.when(i > 0)
    def _():
        c0_ref[...] += p0
        c1_ref[...] += p1


def _scale_body(total, c0_ref, c1_ref, x_ref, o_ref):
    c0 = jnp.sum(c0_ref[...])
    c1 = jnp.sum(c1_ref[...])
    scale = (c1 - c0) / (total - c0)
    o_ref[...] = x_ref[...] * scale


def kernel(x):
    orig_shape = x.shape
    total = x.size
    rows = total // _COLS
    xf = x.reshape(rows, _COLS)

    blk1 = 1024
    counts = pl.pallas_call(
        _count_body,
        grid=(rows // blk1,),
        in_specs=[pl.BlockSpec((blk1, _COLS), lambda i: (i, 0))],
        out_specs=[
            pl.BlockSpec((1, _COLS), lambda i: (0, 0)),
            pl.BlockSpec((1, _COLS), lambda i: (0, 0)),
        ],
        out_shape=[
            jax.ShapeDtypeStruct((1, _COLS), jnp.float32),
            jax.ShapeDtypeStruct((1, _COLS), jnp.float32),
        ],
    )(xf)

    blk2 = 512
    out = pl.pallas_call(
        lambda c0, c1, xr, o: _scale_body(float(total), c0, c1, xr, o),
        grid=(rows // blk2,),
        in_specs=[
            pl.BlockSpec((1, _COLS), lambda i: (0, 0)),
            pl.BlockSpec((1, _COLS), lambda i: (0, 0)),
            pl.BlockSpec((blk2, _COLS), lambda i: (i, 0)),
        ],
        out_specs=pl.BlockSpec((blk2, _COLS), lambda i: (i, 0)),
        out_shape=jax.ShapeDtypeStruct((rows, _COLS), jnp.float32),
    )(counts[0], counts[1], xf)

    return out.reshape(orig_shape)
